# Initial kernel scaffold; baseline (speedup 1.0000x reference)
import functools

import jax
import jax.numpy as jnp
from jax import lax
from jax.experimental import pallas as pl
from jax.experimental.pallas import tpu as pltpu
from jax.experimental.pallas import tpu_sc as plsc

NC = 2
NS = 16
NW = NC * NS
L = 16
D = 512
B = 16
BLK = 64


def _sc_partial_sums(stack_lengths, features):
    mesh = plsc.VectorSubcoreMesh(core_axis_name="c", subcore_axis_name="s")

    @functools.partial(
        pl.kernel,
        out_type=jax.ShapeDtypeStruct((NC, B, D), jnp.float32),
        mesh=mesh,
        scratch_types=[
            pltpu.VMEM((1, L), jnp.int32),
            pltpu.VMEM((BLK, D), jnp.float32),
            pltpu.VMEM((BLK,), jnp.int32),
            pltpu.VMEM((B + 1, D), jnp.float32),
            pltpu.VMEM_SHARED((B + 1, D), jnp.float32),
        ],
    )
    def k(lens_hbm, feat_hbm, out_hbm, lens_v, buf, idx_v, zbuf, acc_sh):
        cid = lax.axis_index("c")
        sid = lax.axis_index("s")
        wid = cid * NS + sid

        pltpu.sync_copy(lens_hbm, lens_v)
        lens = lens_v[0, :]
        ends = jnp.cumsum(lens)
        iota = lax.iota(jnp.int32, L)
        e = [jnp.max(jnp.where(iota == j, ends, 0)) for j in range(B)]
        total = e[B - 1]

        zero = jnp.zeros((L,), jnp.float32)

        @pl.when(sid == 0)
        def _():
            for r in range(B + 1):
                for c in range(D // L):
                    zbuf[r, pl.ds(c * L, L)] = zero
            pltpu.sync_copy(zbuf, acc_sh)

        plsc.subcore_barrier()

        nblk = (total + BLK - 1) // BLK
        nmine = (nblk - wid + NW - 1) // NW

        def body(i, carry):
            base = (wid + i * NW) * BLK
            pltpu.sync_copy(feat_hbm.at[pl.ds(base, BLK)], buf)
            for q in range(BLK // L):
                rv = base + q * L + iota
                seg = jnp.zeros((L,), jnp.int32)
                for j in range(B):
                    seg = seg + jnp.where(e[j] <= rv, 1, 0)
                idx_v[pl.ds(q * L, L)] = seg
            pltpu.sync_copy(buf, acc_sh.at[idx_v], add=True)
            return carry

        lax.fori_loop(0, nmine, body, 0)

        plsc.subcore_barrier()

        pltpu.sync_copy(acc_sh.at[sid], buf.at[0])
        pltpu.sync_copy(buf.at[0], out_hbm.at[cid, sid])

    return k(stack_lengths, features)


def _tc_combine(stack_lengths, partials):
    def body(lens_ref, p_ref, o_ref):
        for i in range(B):
            ln = lens_ref[0, i].astype(jnp.float32)
            o_ref[pl.ds(i, 1), :] = (
                p_ref[0, pl.ds(i, 1), :] + p_ref[1, pl.ds(i, 1), :]
            ) / ln

    return pl.pallas_call(
        body,
        out_shape=jax.ShapeDtypeStruct((B, D), jnp.float32),
        in_specs=[
            pl.BlockSpec(memory_space=pltpu.SMEM),
            pl.BlockSpec(memory_space=pltpu.VMEM),
        ],
        out_specs=pl.BlockSpec(memory_space=pltpu.VMEM),
    )(stack_lengths, partials)


def kernel(stack_lengths, features):
    partials = _sc_partial_sums(stack_lengths, features)
    return _tc_combine(stack_lengths, partials)


# speedup vs baseline: 3.3648x; 3.3648x over previous
import functools

import jax
import jax.numpy as jnp
from jax import lax
from jax.experimental import pallas as pl
from jax.experimental.pallas import tpu as pltpu
from jax.experimental.pallas import tpu_sc as plsc

NC = 2
NS = 16
NW = NC * NS
L = 16
D = 512
B = 16
BLK = 64


def _sc_partial_sums(stack_lengths, features):
    mesh = plsc.VectorSubcoreMesh(core_axis_name="c", subcore_axis_name="s")

    @functools.partial(
        pl.kernel,
        out_type=jax.ShapeDtypeStruct((NW, B, D), jnp.float32),
        mesh=mesh,
        scratch_types=[
            pltpu.VMEM((1, L), jnp.int32),
            pltpu.VMEM((BLK, D), jnp.float32),
            pltpu.VMEM((B + 1, D), jnp.float32),
        ],
        compiler_params=pltpu.CompilerParams(needs_layout_passes=False),
    )
    def k(lens_hbm, feat_hbm, out_hbm, lens_v, buf, acc):
        cid = lax.axis_index("c")
        sid = lax.axis_index("s")
        wid = cid * NS + sid

        pltpu.sync_copy(lens_hbm, lens_v)
        lens = lens_v[0, :]
        ends = jnp.cumsum(lens)
        iota = lax.iota(jnp.int32, L)
        e = [jnp.max(jnp.where(iota == j, ends, 0)) for j in range(B)]
        total = e[B - 1]

        zero = jnp.zeros((L,), jnp.float32)
        for r in range(B + 1):
            for c in range(D // L):
                acc[r, pl.ds(c * L, L)] = zero

        nblk = (total + BLK - 1) // BLK
        nmine = (nblk - wid + NW - 1) // NW

        def blk_body(i, carry):
            base = (wid + i * NW) * BLK
            pltpu.sync_copy(feat_hbm.at[pl.ds(base, BLK)], buf)

            def row_body(r, carry2):
                row = base + r
                s = jnp.int32(0)
                for j in range(B):
                    s = s + jnp.where(e[j] <= row, 1, 0)
                for c in range(D // L):
                    plsc.addupdate(
                        acc.at[s, pl.ds(c * L, L)], buf[r, pl.ds(c * L, L)]
                    )
                return carry2

            lax.fori_loop(0, BLK, row_body, 0)
            return carry

        lax.fori_loop(0, nmine, blk_body, 0)

        pltpu.sync_copy(acc.at[pl.ds(0, B)], out_hbm.at[wid])

    return k(stack_lengths, features)


def _tc_combine(stack_lengths, partials):
    def body(lens_ref, p_ref, o_ref):
        s = p_ref[0]
        for w in range(1, NW):
            s = s + p_ref[w]
        for i in range(B):
            ln = lens_ref[0, i].astype(jnp.float32)
            o_ref[pl.ds(i, 1), :] = s[i : i + 1, :] / ln

    return pl.pallas_call(
        body,
        out_shape=jax.ShapeDtypeStruct((B, D), jnp.float32),
        in_specs=[
            pl.BlockSpec(memory_space=pltpu.SMEM),
            pl.BlockSpec(memory_space=pltpu.VMEM),
        ],
        out_specs=pl.BlockSpec(memory_space=pltpu.VMEM),
    )(stack_lengths, partials)


def kernel(stack_lengths, features):
    partials = _sc_partial_sums(stack_lengths, features)
    return _tc_combine(stack_lengths, partials)


# speedup vs baseline: 3.9686x; 1.1794x over previous
import functools

import jax
import jax.numpy as jnp
from jax import lax
from jax.experimental import pallas as pl
from jax.experimental.pallas import tpu as pltpu
from jax.experimental.pallas import tpu_sc as plsc

NC = 2
NS = 16
NW = NC * NS
L = 16
D = 512
B = 16
BLK = 64


def _sc_partial_sums(stack_lengths, features):
    mesh = plsc.VectorSubcoreMesh(core_axis_name="c", subcore_axis_name="s")

    @functools.partial(
        pl.kernel,
        out_type=jax.ShapeDtypeStruct((NW, B, D), jnp.float32),
        mesh=mesh,
        scratch_types=[
            pltpu.VMEM((1, L), jnp.int32),
            pltpu.VMEM((2, BLK, D), jnp.float32),
            pltpu.VMEM((B + 1, D), jnp.float32),
            pltpu.SemaphoreType.DMA,
            pltpu.SemaphoreType.DMA,
        ],
        compiler_params=pltpu.CompilerParams(needs_layout_passes=False),
    )
    def k(lens_hbm, feat_hbm, out_hbm, lens_v, bufs, acc, sem0, sem1):
        cid = lax.axis_index("c")
        sid = lax.axis_index("s")
        wid = cid * NS + sid

        pltpu.sync_copy(lens_hbm, lens_v)
        lens = lens_v[0, :]
        ends = jnp.cumsum(lens)
        iota = lax.iota(jnp.int32, L)
        e = [jnp.max(jnp.where(iota == j, ends, 0)) for j in range(B)]
        total = e[B - 1]

        zero = jnp.zeros((L,), jnp.float32)
        for r in range(B + 1):
            for c in range(D // L):
                acc[r, pl.ds(c * L, L)] = zero

        nblk = (total + BLK - 1) // BLK
        nmine = (nblk - wid + NW - 1) // NW
        sems = (sem0, sem1)

        def blk_base(i):
            return (wid + i * NW) * BLK

        for b in range(2):
            @pl.when(nmine > b)
            def _(b=b):
                pltpu.async_copy(
                    feat_hbm.at[pl.ds(blk_base(b), BLK)], bufs.at[b], sems[b]
                )

        def consume(i, buf):
            base = blk_base(i)

            def row_body(r, carry2):
                row = base + r
                s = jnp.int32(0)
                for j in range(B):
                    s = s + jnp.where(e[j] <= row, 1, 0)
                for c in range(D // L):
                    plsc.addupdate(
                        acc.at[s, pl.ds(c * L, L)], buf[r, pl.ds(c * L, L)]
                    )
                return carry2

            lax.fori_loop(0, BLK, row_body, 0)

        def pair_body(p, carry):
            for b in range(2):
                i = 2 * p + b

                @pl.when(i < nmine)
                def _(i=i, b=b):
                    pltpu.make_async_copy(
                        feat_hbm.at[pl.ds(0, BLK)], bufs.at[b], sems[b]
                    ).wait()
                    consume(i, bufs.at[b])

                    @pl.when(i + 2 < nmine)
                    def _():
                        pltpu.async_copy(
                            feat_hbm.at[pl.ds(blk_base(i + 2), BLK)],
                            bufs.at[b],
                            sems[b],
                        )
            return carry

        lax.fori_loop(0, (nmine + 1) // 2, pair_body, 0)

        pltpu.sync_copy(acc.at[pl.ds(0, B)], out_hbm.at[wid])

    return k(stack_lengths, features)


def _tc_combine(stack_lengths, partials):
    def body(lens_ref, p_ref, o_ref):
        s = p_ref[0]
        for w in range(1, NW):
            s = s + p_ref[w]
        for i in range(B):
            ln = lens_ref[0, i].astype(jnp.float32)
            o_ref[pl.ds(i, 1), :] = s[i : i + 1, :] / ln

    return pl.pallas_call(
        body,
        out_shape=jax.ShapeDtypeStruct((B, D), jnp.float32),
        in_specs=[
            pl.BlockSpec(memory_space=pltpu.SMEM),
            pl.BlockSpec(memory_space=pltpu.VMEM),
        ],
        out_specs=pl.BlockSpec(memory_space=pltpu.VMEM),
    )(stack_lengths, partials)


def kernel(stack_lengths, features):
    partials = _sc_partial_sums(stack_lengths, features)
    return _tc_combine(stack_lengths, partials)


# speedup vs baseline: 8.4601x; 2.1318x over previous
import functools

import jax
import jax.numpy as jnp
from jax import lax
from jax.experimental import pallas as pl
from jax.experimental.pallas import tpu as pltpu
from jax.experimental.pallas import tpu_sc as plsc

NC = 2
NS = 16
NW = NC * NS
L = 16
D = 512
B = 16
BLK = 64


def _sc_partial_sums(stack_lengths, features):
    mesh = plsc.VectorSubcoreMesh(core_axis_name="c", subcore_axis_name="s")

    @functools.partial(
        pl.kernel,
        out_type=jax.ShapeDtypeStruct((NW, B, D), jnp.float32),
        mesh=mesh,
        scratch_types=[
            pltpu.VMEM((1, L), jnp.int32),
            pltpu.VMEM((2, BLK, D), jnp.float32),
            pltpu.VMEM((B + 1, D), jnp.float32),
            pltpu.SemaphoreType.DMA,
            pltpu.SemaphoreType.DMA,
        ],
        compiler_params=pltpu.CompilerParams(needs_layout_passes=False),
    )
    def k(lens_hbm, feat_hbm, out_hbm, lens_v, bufs, acc, sem0, sem1):
        cid = lax.axis_index("c")
        sid = lax.axis_index("s")
        wid = cid * NS + sid

        pltpu.sync_copy(lens_hbm, lens_v)
        lens = lens_v[0, :]
        ends = jnp.cumsum(lens)
        iota = lax.iota(jnp.int32, L)
        e = [jnp.max(jnp.where(iota == j, ends, 0)) for j in range(B)]
        total = e[B - 1]

        zero = jnp.zeros((L,), jnp.float32)
        for r in range(B + 1):
            for c in range(D // L):
                acc[r, pl.ds(c * L, L)] = zero

        nblk = (total + BLK - 1) // BLK
        nmine = (nblk - wid + NW - 1) // NW
        sems = (sem0, sem1)

        def blk_base(i):
            return (wid + i * NW) * BLK

        for b in range(2):
            @pl.when(nmine > b)
            def _(b=b):
                pltpu.async_copy(
                    feat_hbm.at[pl.ds(blk_base(b), BLK)], bufs.at[b], sems[b]
                )

        def consume(i, buf):
            base = blk_base(i)

            def run_cond(st):
                return st[0] < BLK

            def run_body(st):
                r = st[0]
                row = base + r
                s = jnp.int32(0)
                re = jnp.int32(BLK)
                for j in range(B):
                    in_seg = e[j] <= row
                    s = s + jnp.where(in_seg, 1, 0)
                    ej_rel = e[j] - base
                    re = jnp.where(
                        jnp.logical_and(e[j] > row, ej_rel < re), ej_rel, re
                    )
                for h in range(2):
                    col0 = h * (D // 2)
                    nch = D // 2 // L

                    def inner(rr, vs):
                        return tuple(
                            vs[c] + buf[rr, pl.ds(col0 + c * L, L)]
                            for c in range(nch)
                        )

                    init = tuple(
                        jnp.zeros((L,), jnp.float32) for _ in range(nch)
                    )
                    vs = lax.fori_loop(r, re, inner, init)
                    for c in range(nch):
                        plsc.addupdate(
                            acc.at[s, pl.ds(col0 + c * L, L)], vs[c]
                        )
                return (re,)

            lax.while_loop(run_cond, run_body, (jnp.int32(0),))

        def pair_body(p, carry):
            for b in range(2):
                i = 2 * p + b

                @pl.when(i < nmine)
                def _(i=i, b=b):
                    pltpu.make_async_copy(
                        feat_hbm.at[pl.ds(0, BLK)], bufs.at[b], sems[b]
                    ).wait()
                    consume(i, bufs.at[b])

                    @pl.when(i + 2 < nmine)
                    def _():
                        pltpu.async_copy(
                            feat_hbm.at[pl.ds(blk_base(i + 2), BLK)],
                            bufs.at[b],
                            sems[b],
                        )
            return carry

        lax.fori_loop(0, (nmine + 1) // 2, pair_body, 0)

        pltpu.sync_copy(acc.at[pl.ds(0, B)], out_hbm.at[wid])

    return k(stack_lengths, features)


def _tc_combine(stack_lengths, partials):
    def body(lens_ref, p_ref, o_ref):
        s = p_ref[0]
        for w in range(1, NW):
            s = s + p_ref[w]
        for i in range(B):
            ln = lens_ref[0, i].astype(jnp.float32)
            o_ref[pl.ds(i, 1), :] = s[i : i + 1, :] / ln

    return pl.pallas_call(
        body,
        out_shape=jax.ShapeDtypeStruct((B, D), jnp.float32),
        in_specs=[
            pl.BlockSpec(memory_space=pltpu.SMEM),
            pl.BlockSpec(memory_space=pltpu.VMEM),
        ],
        out_specs=pl.BlockSpec(memory_space=pltpu.VMEM),
    )(stack_lengths, partials)


def kernel(stack_lengths, features):
    partials = _sc_partial_sums(stack_lengths, features)
    return _tc_combine(stack_lengths, partials)
